# baseline (speedup 1.0000x reference)
import functools

import jax
import jax.numpy as jnp
from jax import lax
from jax.experimental import pallas as pl
from jax.experimental.pallas import tpu as pltpu
from jax.experimental.pallas import tpu_sc as plsc

N = 320000
C = 128
NUM_SEGMENTS = 16
EPS = 1e-6
BR = 16000

SC_CORES = 2
SC_SUBCORES = 16
SC_WORKERS = SC_CORES * SC_SUBCORES
SC_CHUNK = N // SC_WORKERS
SC_LANES = 16


def _sc_counts_kernel(seg_hbm, out_hbm, ids_v, acc_v):
    wid = lax.axis_index("s") * SC_CORES + lax.axis_index("c")
    base = wid * SC_CHUNK
    pltpu.sync_copy(seg_hbm.at[pl.ds(base, SC_CHUNK)], ids_v)
    for t in range(NUM_SEGMENTS):
        acc_v[t] = jnp.zeros((SC_LANES,), jnp.int32)

    def body(i, _):
        v = ids_v[pl.ds(i * SC_LANES, SC_LANES)]
        for t in range(NUM_SEGMENTS):
            tv = jnp.full((SC_LANES,), t, jnp.int32)
            acc_v[t] = acc_v[t] + jnp.where(v == tv, 1, 0).astype(jnp.int32)
        return 0

    lax.fori_loop(0, SC_CHUNK // SC_LANES, body, 0)
    pltpu.sync_copy(acc_v, out_hbm.at[wid])


_sc_counts = functools.partial(
    pl.kernel,
    out_type=jax.ShapeDtypeStruct((SC_WORKERS, NUM_SEGMENTS, SC_LANES),
                                  jnp.int32),
    scratch_types=[
        pltpu.VMEM((SC_CHUNK,), jnp.int32),
        pltpu.VMEM((NUM_SEGMENTS, SC_LANES), jnp.int32),
    ],
    mesh=plsc.VectorSubcoreMesh(core_axis_name="c", subcore_axis_name="s"),
)(_sc_counts_kernel)


def _onehot_t(seg_ref):
    seg = seg_ref[0, 0, :]
    segb = jnp.broadcast_to(seg[None, :], (NUM_SEGMENTS, BR))
    tid = jax.lax.broadcasted_iota(jnp.int32, (NUM_SEGMENTS, BR), 0)
    return (segb == tid).astype(jnp.float32)


def _stats_kernel(x_ref, seg_ref, sums_ref):
    @pl.when(pl.program_id(0) == 0)
    def _():
        sums_ref[...] = jnp.zeros_like(sums_ref)

    xb = x_ref[...]
    oh = _onehot_t(seg_ref)
    ps = jax.lax.dot_general(
        oh, xb, (((1,), (0,)), ((), ())),
        preferred_element_type=jnp.float32)
    pq = jax.lax.dot_general(
        oh, xb * xb, (((1,), (0,)), ((), ())),
        preferred_element_type=jnp.float32)
    sums_ref[:, :C] += ps
    sums_ref[:, C:] += pq


def _norm_kernel(x_ref, seg_ref, stats_ref, cparts_ref, w_ref, b_ref,
                 out_ref):
    counts = jnp.sum(cparts_ref[...].astype(jnp.float32), axis=1)
    cnt = jnp.maximum(counts, 1.0)[:, None]
    inv = 1.0 / cnt
    mean = stats_ref[:, :C] * inv
    msq = stats_ref[:, C:] * inv
    var = msq - mean * mean
    instd = jax.lax.rsqrt(var + EPS)
    scale = instd * w_ref[...]
    shift = b_ref[...] - mean * scale
    st = jnp.concatenate([scale, shift], axis=1)

    oh = _onehot_t(seg_ref)
    ST = jax.lax.dot_general(
        oh, st, (((0,), (0,)), ((), ())),
        preferred_element_type=jnp.float32)
    out_ref[...] = x_ref[...] * ST[:, :C] + ST[:, C:]


def kernel(x, segment_ids, weight, bias):
    seg_i32 = segment_ids.astype(jnp.int32)
    seg3d = seg_i32.reshape(N // BR, 1, BR)
    grid = (N // BR,)

    cparts = _sc_counts(seg_i32)
    cparts_t = cparts.transpose(1, 0, 2).reshape(NUM_SEGMENTS,
                                                 SC_WORKERS * SC_LANES)

    stats = pl.pallas_call(
        _stats_kernel,
        grid=grid,
        in_specs=[
            pl.BlockSpec((BR, C), lambda i: (i, 0)),
            pl.BlockSpec((1, 1, BR), lambda i: (i, 0, 0)),
        ],
        out_specs=pl.BlockSpec((NUM_SEGMENTS, 2 * C), lambda i: (0, 0)),
        out_shape=jax.ShapeDtypeStruct((NUM_SEGMENTS, 2 * C), jnp.float32),
    )(x, seg3d)

    out = pl.pallas_call(
        _norm_kernel,
        grid=grid,
        in_specs=[
            pl.BlockSpec((BR, C), lambda i: (i, 0)),
            pl.BlockSpec((1, 1, BR), lambda i: (i, 0, 0)),
            pl.BlockSpec((NUM_SEGMENTS, 2 * C), lambda i: (0, 0)),
            pl.BlockSpec((NUM_SEGMENTS, SC_WORKERS * SC_LANES),
                         lambda i: (0, 0)),
            pl.BlockSpec((1, C), lambda i: (0, 0)),
            pl.BlockSpec((1, C), lambda i: (0, 0)),
        ],
        out_specs=pl.BlockSpec((BR, C), lambda i: (i, 0)),
        out_shape=jax.ShapeDtypeStruct((N, C), jnp.float32),
    )(x, seg3d, stats, cparts_t, weight, bias)
    return out
